# Initial kernel scaffold; baseline (speedup 1.0000x reference)
import jax
import jax.numpy as jnp
from jax.experimental import pallas as pl


def kernel(adj_indices, adj_values, user_emb, item_id_emb, brand_emb, content_emb, W, b):
    raise NotImplementedError("write your pallas kernel here")


# speedup vs baseline: 9.3812x; 9.3812x over previous
import functools

import jax
import jax.numpy as jnp
from jax import lax
from jax.experimental import pallas as pl
from jax.experimental.pallas import tpu as pltpu
from jax.experimental.pallas import tpu_sc as plsc

NC = 2
NS = 16
LANES = 16
GROUP = 128
GPB = 8
BLK = GROUP * GPB


@functools.lru_cache(maxsize=None)
def _spmm_sc(n_nodes: int, nb: int):
    nfull = n_nodes // GROUP
    ztail = n_nodes % GROUP
    zbase_cnt, zrem = nfull // NS, nfull % NS
    mesh = plsc.VectorSubcoreMesh(core_axis_name="c", subcore_axis_name="s",
                                  num_cores=NC, num_subcores=NS)
    out_t = (jax.ShapeDtypeStruct((n_nodes, LANES), jnp.float32),
             jax.ShapeDtypeStruct((n_nodes, LANES), jnp.float32))

    def body(ego_lo, ego_hi, src2d, dst2d, vals2d, out_lo, out_hi,
             src_v, dst_v, vals_v, rows_v, zeros_v, acc, sem):
        cid = lax.axis_index("c")
        sid = lax.axis_index("s")

        def run(ego, out):
            def zrow(r, c):
                zeros_v[r] = jnp.zeros((LANES,), jnp.float32)
                return c
            lax.fori_loop(0, GROUP, zrow, 0)
            ntrip = zbase_cnt + (sid < zrem).astype(jnp.int32)

            def zchunk(c, carry):
                off = pl.multiple_of((sid + c * NS) * GROUP, GROUP)
                pltpu.sync_copy(zeros_v, acc.at[pl.ds(off, GROUP)])
                return carry
            lax.fori_loop(0, ntrip, zchunk, 0)
            if ztail:
                @pl.when(sid == NS - 1)
                def _():
                    pltpu.sync_copy(zeros_v.at[pl.ds(0, ztail)],
                                    acc.at[pl.ds(nfull * GROUP, ztail)])
            plsc.subcore_barrier()

            rbase = sid * (nb * GPB)

            def block(bk, carry):
                r0 = rbase + bk * GPB
                pltpu.sync_copy(src2d.at[pl.ds(r0, GPB)], src_v)
                pltpu.sync_copy(dst2d.at[pl.ds(r0, GPB)], dst_v)
                pltpu.sync_copy(vals2d.at[pl.ds(r0, GPB)], vals_v)

                def fire(g, c):
                    pltpu.async_copy(ego.at[src_v.at[g]],
                                     rows_v.at[pl.ds(g * GROUP, GROUP)], sem)
                    return c
                lax.fori_loop(0, GPB, fire, 0)

                def drain(g, c):
                    pltpu.make_async_copy(
                        ego.at[src_v.at[g]],
                        rows_v.at[pl.ds(g * GROUP, GROUP)], sem).wait()
                    return c
                lax.fori_loop(0, GPB, drain, 0)

                def scale(g, c):
                    def sixteen(e8, c2):
                        e0 = e8 * LANES
                        base = g * GROUP + e0
                        vv = vals_v[g, pl.ds(e0, LANES)]
                        for k in range(LANES):
                            rows_v[base + k] = rows_v[base + k] * vv[k]
                        return c2
                    lax.fori_loop(0, GROUP // LANES, sixteen, 0)
                    return c
                lax.fori_loop(0, GPB, scale, 0)

                def scat(g, c):
                    pltpu.sync_copy(rows_v.at[pl.ds(g * GROUP, GROUP)],
                                    acc.at[dst_v.at[g]], add=True)
                    return c
                lax.fori_loop(0, GPB, scat, 0)
                return carry
            lax.fori_loop(0, nb, block, 0)

            plsc.subcore_barrier()

            def fchunk(c, carry):
                off = pl.multiple_of((sid + c * NS) * GROUP, GROUP)
                pltpu.sync_copy(acc.at[pl.ds(off, GROUP)],
                                out.at[pl.ds(off, GROUP)])
                return carry
            lax.fori_loop(0, ntrip, fchunk, 0)
            if ztail:
                @pl.when(sid == NS - 1)
                def _():
                    off = nfull * GROUP
                    pltpu.sync_copy(acc.at[pl.ds(off, ztail)],
                                    out.at[pl.ds(off, ztail)])

        @pl.when(cid == 0)
        def _():
            run(ego_lo, out_lo)

        @pl.when(cid == 1)
        def _():
            run(ego_hi, out_hi)

    return pl.kernel(
        body, out_type=out_t, mesh=mesh,
        compiler_params=pltpu.CompilerParams(use_tc_tiling_on_sc=False),
        scratch_types=[
            pltpu.VMEM((GPB, GROUP), jnp.int32),
            pltpu.VMEM((GPB, GROUP), jnp.int32),
            pltpu.VMEM((GPB, GROUP), jnp.float32),
            pltpu.VMEM((BLK, LANES), jnp.float32),
            pltpu.VMEM((GROUP, LANES), jnp.float32),
            pltpu.VMEM_SHARED((n_nodes, LANES), jnp.float32),
            pltpu.SemaphoreType.DMA,
        ],
    )


def _fusion(item_id_emb, content_emb, w1t, w2t, b2):
    m, d = item_id_emb.shape
    cd = content_emb.shape[1]
    blk = 2000
    assert m % blk == 0

    def body(x1, x2, w1, w2, bb, o):
        acc = jnp.dot(x1[...], w1[...], preferred_element_type=jnp.float32)
        acc = acc + jnp.dot(x2[...], w2[...], preferred_element_type=jnp.float32)
        acc = acc + bb[...]
        o[...] = jnp.where(acc >= 0, acc, 0.01 * acc)

    return pl.pallas_call(
        body,
        grid=(m // blk,),
        in_specs=[pl.BlockSpec((blk, d), lambda i: (i, 0)),
                  pl.BlockSpec((blk, cd), lambda i: (i, 0)),
                  pl.BlockSpec((d, d), lambda i: (0, 0)),
                  pl.BlockSpec((cd, d), lambda i: (0, 0)),
                  pl.BlockSpec((1, d), lambda i: (0, 0))],
        out_specs=pl.BlockSpec((blk, d), lambda i: (i, 0)),
        out_shape=jax.ShapeDtypeStruct((m, d), jnp.float32),
    )(item_id_emb, content_emb, w1t, w2t, b2)


def _combine(a0, a1, a2, a3, b0, b1, b2, b3):
    n, h = a0.shape
    blk = 2000
    assert n % blk == 0

    def body(r0, r1, r2, r3, s0, s1, s2, s3, olo, ohi):
        olo[...] = (r0[...] + r1[...] + r2[...] + r3[...]) * 0.25
        ohi[...] = (s0[...] + s1[...] + s2[...] + s3[...]) * 0.25

    spec = pl.BlockSpec((blk, h), lambda i: (i, 0))
    return pl.pallas_call(
        body,
        grid=(n // blk,),
        in_specs=[spec] * 8,
        out_specs=(spec, spec),
        out_shape=(jax.ShapeDtypeStruct((n, h), jnp.float32),
                   jax.ShapeDtypeStruct((n, h), jnp.float32)),
    )(a0, a1, a2, a3, b0, b1, b2, b3)


def kernel(adj_indices, adj_values, user_emb, item_id_emb, brand_emb,
           content_emb, W, b):
    nu, d = user_emb.shape
    ni = item_id_emb.shape[0]
    nbrand = brand_emb.shape[0]
    n = nu + ni + nbrand
    e = adj_values.shape[0]
    half = d // 2

    idx = adj_indices.astype(jnp.int32)
    dstv, srcv = idx[0], idx[1]
    vals = adj_values.astype(jnp.float32)

    blk_edges = NS * BLK
    nb = -(-e // blk_edges)
    pad = nb * blk_edges - e
    src2d = jnp.pad(srcv, (0, pad)).reshape(-1, GROUP)
    dst2d = jnp.pad(dstv, (0, pad)).reshape(-1, GROUP)
    vals2d = jnp.pad(vals, (0, pad)).reshape(-1, GROUP)

    fused = _fusion(item_id_emb, content_emb,
                    jnp.transpose(W[:, :d]), jnp.transpose(W[:, d:]),
                    b.reshape(1, d))
    ego0 = jnp.concatenate([user_emb, fused, brand_emb], axis=0)
    e0lo, e0hi = ego0[:, :half], ego0[:, half:]

    spmm = _spmm_sc(n, nb)
    l1lo, l1hi = spmm(e0lo, e0hi, src2d, dst2d, vals2d)
    l2lo, l2hi = spmm(l1lo, l1hi, src2d, dst2d, vals2d)
    l3lo, l3hi = spmm(l2lo, l2hi, src2d, dst2d, vals2d)

    flo, fhi = _combine(e0lo, l1lo, l2lo, l3lo, e0hi, l1hi, l2hi, l3hi)
    final = jnp.concatenate([flo, fhi], axis=1)
    return (final[:nu], final[nu:nu + ni], final[nu + ni:],
            user_emb, item_id_emb)


# speedup vs baseline: 15.3198x; 1.6330x over previous
import functools

import jax
import jax.numpy as jnp
from jax import lax
from jax.experimental import pallas as pl
from jax.experimental.pallas import tpu as pltpu
from jax.experimental.pallas import tpu_sc as plsc

NC = 2
NS = 16
LANES = 16
GROUP = 128
GPB = 8
BLK = GROUP * GPB


@functools.lru_cache(maxsize=None)
def _spmm_sc(n_nodes: int, nb: int):
    nfull = n_nodes // GROUP
    ztail = n_nodes % GROUP
    zbase_cnt, zrem = nfull // NS, nfull % NS
    mesh = plsc.VectorSubcoreMesh(core_axis_name="c", subcore_axis_name="s",
                                  num_cores=NC, num_subcores=NS)
    out_t = (jax.ShapeDtypeStruct((n_nodes, LANES), jnp.float32),
             jax.ShapeDtypeStruct((n_nodes, LANES), jnp.float32))

    def body(ego_lo, ego_hi, src2d, dst2d, vals2d, out_lo, out_hi,
             src_v, dst_v, vals_v, rows_v, acc, gsem, ssem, isem):
        cid = lax.axis_index("c")
        sid = lax.axis_index("s")

        def run(ego, out):
            def zrow(r, c):
                rows_v[r] = jnp.zeros((LANES,), jnp.float32)
                return c
            lax.fori_loop(0, GROUP, zrow, 0)
            ntrip = zbase_cnt + (sid < zrem).astype(jnp.int32)

            def zchunk(c, carry):
                off = pl.multiple_of((sid + c * NS) * GROUP, GROUP)
                pltpu.sync_copy(rows_v.at[pl.ds(0, GROUP)],
                                acc.at[pl.ds(off, GROUP)])
                return carry
            lax.fori_loop(0, ntrip, zchunk, 0)
            if ztail:
                @pl.when(sid == NS - 1)
                def _():
                    pltpu.sync_copy(rows_v.at[pl.ds(0, ztail)],
                                    acc.at[pl.ds(nfull * GROUP, ztail)])
            plsc.subcore_barrier()

            rbase = sid * (nb * GPB)
            pltpu.sync_copy(src2d.at[pl.ds(rbase, GPB)], src_v.at[0])
            pltpu.sync_copy(dst2d.at[pl.ds(rbase, GPB)], dst_v.at[0])
            pltpu.sync_copy(vals2d.at[pl.ds(rbase, GPB)], vals_v.at[0])

            def rows_g(g):
                return rows_v.at[pl.ds(g * GROUP, GROUP)]

            def block(bk, carry):
                par = lax.rem(bk, 2)
                nxt = 1 - par

                @pl.when(bk + 1 < nb)
                def _():
                    r1 = rbase + (bk + 1) * GPB
                    pltpu.async_copy(src2d.at[pl.ds(r1, GPB)], src_v.at[nxt], isem)
                    pltpu.async_copy(dst2d.at[pl.ds(r1, GPB)], dst_v.at[nxt], isem)
                    pltpu.async_copy(vals2d.at[pl.ds(r1, GPB)], vals_v.at[nxt], isem)

                @pl.when(bk > 0)
                def _():
                    for g in range(GPB):
                        pltpu.make_async_copy(
                            rows_g(g), acc.at[dst_v.at[nxt, g]],
                            ssem.at[g]).wait()

                for g in range(GPB):
                    pltpu.async_copy(ego.at[src_v.at[par, g]], rows_g(g),
                                     gsem.at[g])
                for g in range(GPB):
                    pltpu.make_async_copy(ego.at[src_v.at[par, g]], rows_g(g),
                                          gsem.at[g]).wait()

                    def sixteen(e8, c2):
                        e0 = e8 * LANES
                        base = g * GROUP + e0
                        vv = vals_v[par, g, pl.ds(e0, LANES)]
                        for k in range(LANES):
                            rows_v[base + k] = rows_v[base + k] * vv[k]
                        return c2
                    lax.fori_loop(0, GROUP // LANES, sixteen, 0)
                    pltpu.async_copy(rows_g(g), acc.at[dst_v.at[par, g]],
                                     ssem.at[g], add=True)

                @pl.when(bk + 1 < nb)
                def _():
                    r1 = rbase + (bk + 1) * GPB
                    pltpu.make_async_copy(src2d.at[pl.ds(r1, GPB)],
                                          src_v.at[nxt], isem).wait()
                    pltpu.make_async_copy(dst2d.at[pl.ds(r1, GPB)],
                                          dst_v.at[nxt], isem).wait()
                    pltpu.make_async_copy(vals2d.at[pl.ds(r1, GPB)],
                                          vals_v.at[nxt], isem).wait()
                return carry
            lax.fori_loop(0, nb, block, 0)

            lastpar = lax.rem(jnp.int32(nb - 1), 2)
            for g in range(GPB):
                pltpu.make_async_copy(rows_g(g), acc.at[dst_v.at[lastpar, g]],
                                      ssem.at[g]).wait()
            plsc.subcore_barrier()

            def fchunk(c, carry):
                off = pl.multiple_of((sid + c * NS) * GROUP, GROUP)
                pltpu.sync_copy(acc.at[pl.ds(off, GROUP)],
                                out.at[pl.ds(off, GROUP)])
                return carry
            lax.fori_loop(0, ntrip, fchunk, 0)
            if ztail:
                @pl.when(sid == NS - 1)
                def _():
                    off = nfull * GROUP
                    pltpu.sync_copy(acc.at[pl.ds(off, ztail)],
                                    out.at[pl.ds(off, ztail)])

        @pl.when(cid == 0)
        def _():
            run(ego_lo, out_lo)

        @pl.when(cid == 1)
        def _():
            run(ego_hi, out_hi)

    return pl.kernel(
        body, out_type=out_t, mesh=mesh,
        compiler_params=pltpu.CompilerParams(use_tc_tiling_on_sc=False),
        scratch_types=[
            pltpu.VMEM((2, GPB, GROUP), jnp.int32),
            pltpu.VMEM((2, GPB, GROUP), jnp.int32),
            pltpu.VMEM((2, GPB, GROUP), jnp.float32),
            pltpu.VMEM((BLK, LANES), jnp.float32),
            pltpu.VMEM_SHARED((n_nodes, LANES), jnp.float32),
            pltpu.SemaphoreType.DMA((GPB,)),
            pltpu.SemaphoreType.DMA((GPB,)),
            pltpu.SemaphoreType.DMA,
        ],
    )


def _fusion(item_id_emb, content_emb, w1t, w2t, b2):
    m, d = item_id_emb.shape
    cd = content_emb.shape[1]
    blk = 2000
    assert m % blk == 0

    def body(x1, x2, w1, w2, bb, o):
        acc = jnp.dot(x1[...], w1[...], preferred_element_type=jnp.float32)
        acc = acc + jnp.dot(x2[...], w2[...], preferred_element_type=jnp.float32)
        acc = acc + bb[...]
        o[...] = jnp.where(acc >= 0, acc, 0.01 * acc)

    return pl.pallas_call(
        body,
        grid=(m // blk,),
        in_specs=[pl.BlockSpec((blk, d), lambda i: (i, 0)),
                  pl.BlockSpec((blk, cd), lambda i: (i, 0)),
                  pl.BlockSpec((d, d), lambda i: (0, 0)),
                  pl.BlockSpec((cd, d), lambda i: (0, 0)),
                  pl.BlockSpec((1, d), lambda i: (0, 0))],
        out_specs=pl.BlockSpec((blk, d), lambda i: (i, 0)),
        out_shape=jax.ShapeDtypeStruct((m, d), jnp.float32),
    )(item_id_emb, content_emb, w1t, w2t, b2)


def _combine(a0, a1, a2, a3, b0, b1, b2, b3):
    n, h = a0.shape
    blk = 2000
    assert n % blk == 0

    def body(r0, r1, r2, r3, s0, s1, s2, s3, olo, ohi):
        olo[...] = (r0[...] + r1[...] + r2[...] + r3[...]) * 0.25
        ohi[...] = (s0[...] + s1[...] + s2[...] + s3[...]) * 0.25

    spec = pl.BlockSpec((blk, h), lambda i: (i, 0))
    return pl.pallas_call(
        body,
        grid=(n // blk,),
        in_specs=[spec] * 8,
        out_specs=(spec, spec),
        out_shape=(jax.ShapeDtypeStruct((n, h), jnp.float32),
                   jax.ShapeDtypeStruct((n, h), jnp.float32)),
    )(a0, a1, a2, a3, b0, b1, b2, b3)


def kernel(adj_indices, adj_values, user_emb, item_id_emb, brand_emb,
           content_emb, W, b):
    nu, d = user_emb.shape
    ni = item_id_emb.shape[0]
    nbrand = brand_emb.shape[0]
    n = nu + ni + nbrand
    e = adj_values.shape[0]
    half = d // 2

    idx = adj_indices.astype(jnp.int32)
    dstv, srcv = idx[0], idx[1]
    vals = adj_values.astype(jnp.float32)

    blk_edges = NS * BLK
    nb = -(-e // blk_edges)
    pad = nb * blk_edges - e
    src2d = jnp.pad(srcv, (0, pad)).reshape(-1, GROUP)
    dst2d = jnp.pad(dstv, (0, pad)).reshape(-1, GROUP)
    vals2d = jnp.pad(vals, (0, pad)).reshape(-1, GROUP)

    fused = _fusion(item_id_emb, content_emb,
                    jnp.transpose(W[:, :d]), jnp.transpose(W[:, d:]),
                    b.reshape(1, d))
    ego0 = jnp.concatenate([user_emb, fused, brand_emb], axis=0)
    e0lo, e0hi = ego0[:, :half], ego0[:, half:]

    spmm = _spmm_sc(n, nb)
    l1lo, l1hi = spmm(e0lo, e0hi, src2d, dst2d, vals2d)
    l2lo, l2hi = spmm(l1lo, l1hi, src2d, dst2d, vals2d)
    l3lo, l3hi = spmm(l2lo, l2hi, src2d, dst2d, vals2d)

    flo, fhi = _combine(e0lo, l1lo, l2lo, l3lo, e0hi, l1hi, l2hi, l3hi)
    final = jnp.concatenate([flo, fhi], axis=1)
    return (final[:nu], final[nu:nu + ni], final[nu + ni:],
            user_emb, item_id_emb)


# speedup vs baseline: 15.6250x; 1.0199x over previous
import functools

import jax
import jax.numpy as jnp
from jax import lax
from jax.experimental import pallas as pl
from jax.experimental.pallas import tpu as pltpu
from jax.experimental.pallas import tpu_sc as plsc

NC = 2
NS = 16
LANES = 16
GROUP = 128
GPB = 8
BLK = GROUP * GPB


@functools.lru_cache(maxsize=None)
def _spmm_sc(n_nodes: int, nb: int):
    nfull = n_nodes // GROUP
    ztail = n_nodes % GROUP
    zbase_cnt, zrem = nfull // NS, nfull % NS
    mesh = plsc.VectorSubcoreMesh(core_axis_name="c", subcore_axis_name="s",
                                  num_cores=NC, num_subcores=NS)
    out_t = (jax.ShapeDtypeStruct((n_nodes, LANES), jnp.float32),
             jax.ShapeDtypeStruct((n_nodes, LANES), jnp.float32))

    def body(ego_lo, ego_hi, src2d, dst2d, vals2d, out_lo, out_hi,
             src_v, dst_v, vals_v, rows_v, acc, gsem, ssem, isem):
        cid = lax.axis_index("c")
        sid = lax.axis_index("s")

        def run(ego, out):
            def zrow(r, c):
                rows_v[r] = jnp.zeros((LANES,), jnp.float32)
                return c
            lax.fori_loop(0, GROUP, zrow, 0)
            ntrip = zbase_cnt + (sid < zrem).astype(jnp.int32)

            def zchunk(c, carry):
                off = pl.multiple_of((sid + c * NS) * GROUP, GROUP)
                pltpu.sync_copy(rows_v.at[pl.ds(0, GROUP)],
                                acc.at[pl.ds(off, GROUP)])
                return carry
            lax.fori_loop(0, ntrip, zchunk, 0)
            if ztail:
                @pl.when(sid == NS - 1)
                def _():
                    pltpu.sync_copy(rows_v.at[pl.ds(0, ztail)],
                                    acc.at[pl.ds(nfull * GROUP, ztail)])
            plsc.subcore_barrier()

            rbase = sid * (nb * GPB)
            pltpu.sync_copy(src2d.at[pl.ds(rbase, GPB)], src_v.at[0])
            pltpu.sync_copy(dst2d.at[pl.ds(rbase, GPB)], dst_v.at[0])
            pltpu.sync_copy(vals2d.at[pl.ds(rbase, GPB)], vals_v.at[0])

            def rows_g(g):
                return rows_v.at[pl.ds(g * GROUP, GROUP)]

            def block(bk, carry):
                par = lax.rem(bk, 2)
                nxt = 1 - par

                @pl.when(bk + 1 < nb)
                def _():
                    r1 = rbase + (bk + 1) * GPB
                    pltpu.async_copy(src2d.at[pl.ds(r1, GPB)], src_v.at[nxt], isem)
                    pltpu.async_copy(dst2d.at[pl.ds(r1, GPB)], dst_v.at[nxt], isem)
                    pltpu.async_copy(vals2d.at[pl.ds(r1, GPB)], vals_v.at[nxt], isem)

                @pl.when(bk > 0)
                def _():
                    for g in range(GPB):
                        pltpu.make_async_copy(
                            rows_g(g), acc.at[dst_v.at[nxt, g]],
                            ssem.at[g]).wait()

                for g in range(GPB):
                    pltpu.async_copy(ego.at[src_v.at[par, g]], rows_g(g),
                                     gsem.at[g])
                for g in range(GPB):
                    pltpu.make_async_copy(ego.at[src_v.at[par, g]], rows_g(g),
                                          gsem.at[g]).wait()

                    @plsc.parallel_loop(0, GROUP, step=LANES, unroll=2)
                    def _(e0):
                        base = g * GROUP + e0
                        vv = vals_v[par, g, pl.ds(e0, LANES)]
                        for k in range(LANES):
                            rows_v[base + k] = rows_v[base + k] * vv[k]
                    pltpu.async_copy(rows_g(g), acc.at[dst_v.at[par, g]],
                                     ssem.at[g], add=True)

                @pl.when(bk + 1 < nb)
                def _():
                    r1 = rbase + (bk + 1) * GPB
                    pltpu.make_async_copy(src2d.at[pl.ds(r1, GPB)],
                                          src_v.at[nxt], isem).wait()
                    pltpu.make_async_copy(dst2d.at[pl.ds(r1, GPB)],
                                          dst_v.at[nxt], isem).wait()
                    pltpu.make_async_copy(vals2d.at[pl.ds(r1, GPB)],
                                          vals_v.at[nxt], isem).wait()
                return carry
            lax.fori_loop(0, nb, block, 0)

            lastpar = lax.rem(jnp.int32(nb - 1), 2)
            for g in range(GPB):
                pltpu.make_async_copy(rows_g(g), acc.at[dst_v.at[lastpar, g]],
                                      ssem.at[g]).wait()
            plsc.subcore_barrier()

            def fchunk(c, carry):
                off = pl.multiple_of((sid + c * NS) * GROUP, GROUP)
                pltpu.sync_copy(acc.at[pl.ds(off, GROUP)],
                                out.at[pl.ds(off, GROUP)])
                return carry
            lax.fori_loop(0, ntrip, fchunk, 0)
            if ztail:
                @pl.when(sid == NS - 1)
                def _():
                    off = nfull * GROUP
                    pltpu.sync_copy(acc.at[pl.ds(off, ztail)],
                                    out.at[pl.ds(off, ztail)])

        @pl.when(cid == 0)
        def _():
            run(ego_lo, out_lo)

        @pl.when(cid == 1)
        def _():
            run(ego_hi, out_hi)

    return pl.kernel(
        body, out_type=out_t, mesh=mesh,
        compiler_params=pltpu.CompilerParams(use_tc_tiling_on_sc=False),
        scratch_types=[
            pltpu.VMEM((2, GPB, GROUP), jnp.int32),
            pltpu.VMEM((2, GPB, GROUP), jnp.int32),
            pltpu.VMEM((2, GPB, GROUP), jnp.float32),
            pltpu.VMEM((BLK, LANES), jnp.float32),
            pltpu.VMEM_SHARED((n_nodes, LANES), jnp.float32),
            pltpu.SemaphoreType.DMA((GPB,)),
            pltpu.SemaphoreType.DMA((GPB,)),
            pltpu.SemaphoreType.DMA,
        ],
    )


def _fusion(item_id_emb, content_emb, w1t, w2t, b2):
    m, d = item_id_emb.shape
    cd = content_emb.shape[1]
    blk = 2000
    assert m % blk == 0

    def body(x1, x2, w1, w2, bb, o):
        acc = jnp.dot(x1[...], w1[...], preferred_element_type=jnp.float32)
        acc = acc + jnp.dot(x2[...], w2[...], preferred_element_type=jnp.float32)
        acc = acc + bb[...]
        o[...] = jnp.where(acc >= 0, acc, 0.01 * acc)

    return pl.pallas_call(
        body,
        grid=(m // blk,),
        in_specs=[pl.BlockSpec((blk, d), lambda i: (i, 0)),
                  pl.BlockSpec((blk, cd), lambda i: (i, 0)),
                  pl.BlockSpec((d, d), lambda i: (0, 0)),
                  pl.BlockSpec((cd, d), lambda i: (0, 0)),
                  pl.BlockSpec((1, d), lambda i: (0, 0))],
        out_specs=pl.BlockSpec((blk, d), lambda i: (i, 0)),
        out_shape=jax.ShapeDtypeStruct((m, d), jnp.float32),
    )(item_id_emb, content_emb, w1t, w2t, b2)


def _combine(a0, a1, a2, a3, b0, b1, b2, b3):
    n, h = a0.shape
    blk = 2000
    assert n % blk == 0

    def body(r0, r1, r2, r3, s0, s1, s2, s3, olo, ohi):
        olo[...] = (r0[...] + r1[...] + r2[...] + r3[...]) * 0.25
        ohi[...] = (s0[...] + s1[...] + s2[...] + s3[...]) * 0.25

    spec = pl.BlockSpec((blk, h), lambda i: (i, 0))
    return pl.pallas_call(
        body,
        grid=(n // blk,),
        in_specs=[spec] * 8,
        out_specs=(spec, spec),
        out_shape=(jax.ShapeDtypeStruct((n, h), jnp.float32),
                   jax.ShapeDtypeStruct((n, h), jnp.float32)),
    )(a0, a1, a2, a3, b0, b1, b2, b3)


def kernel(adj_indices, adj_values, user_emb, item_id_emb, brand_emb,
           content_emb, W, b):
    nu, d = user_emb.shape
    ni = item_id_emb.shape[0]
    nbrand = brand_emb.shape[0]
    n = nu + ni + nbrand
    e = adj_values.shape[0]
    half = d // 2

    idx = adj_indices.astype(jnp.int32)
    dstv, srcv = idx[0], idx[1]
    vals = adj_values.astype(jnp.float32)

    blk_edges = NS * BLK
    nb = -(-e // blk_edges)
    pad = nb * blk_edges - e
    src2d = jnp.pad(srcv, (0, pad)).reshape(-1, GROUP)
    dst2d = jnp.pad(dstv, (0, pad)).reshape(-1, GROUP)
    vals2d = jnp.pad(vals, (0, pad)).reshape(-1, GROUP)

    fused = _fusion(item_id_emb, content_emb,
                    jnp.transpose(W[:, :d]), jnp.transpose(W[:, d:]),
                    b.reshape(1, d))
    ego0 = jnp.concatenate([user_emb, fused, brand_emb], axis=0)
    e0lo, e0hi = ego0[:, :half], ego0[:, half:]

    spmm = _spmm_sc(n, nb)
    l1lo, l1hi = spmm(e0lo, e0hi, src2d, dst2d, vals2d)
    l2lo, l2hi = spmm(l1lo, l1hi, src2d, dst2d, vals2d)
    l3lo, l3hi = spmm(l2lo, l2hi, src2d, dst2d, vals2d)

    flo, fhi = _combine(e0lo, l1lo, l2lo, l3lo, e0hi, l1hi, l2hi, l3hi)
    final = jnp.concatenate([flo, fhi], axis=1)
    return (final[:nu], final[nu:nu + ni], final[nu + ni:],
            user_emb, item_id_emb)
